# Initial kernel scaffold; baseline (speedup 1.0000x reference)
import jax
import jax.numpy as jnp
from jax.experimental import pallas as pl


def kernel(W, fisher_forget, fisher_retain):
    raise NotImplementedError("write your pallas kernel here")


# speedup vs baseline: 8.4055x; 8.4055x over previous
import functools

import jax
import jax.numpy as jnp
import numpy as np
from jax import lax
from jax.experimental import pallas as pl
from jax.experimental.pallas import tpu as pltpu
from jax.experimental.pallas import tpu_sc as plsc

ROWS, COLS = 8192, 4096
N = ROWS * COLS
K = N // 10
NC, NS, L = 2, 16, 16
NW = NC * NS
M = N // NW
CHUNK = 8192
VECS = CHUNK // L
NCHUNKS = M // CHUNK
B1 = 4096
B3 = 256
MININT = np.int32(-2147483648)

_mesh = plsc.VectorSubcoreMesh(core_axis_name="c", subcore_axis_name="s")
_params = pltpu.CompilerParams(needs_layout_passes=False)


def _wid():
    return lax.axis_index("s") * NC + lax.axis_index("c")


def _zero_hist(hist_v, nwords):
    def body(i, _):
        hist_v[pl.ds(i * L, L)] = jnp.zeros((L,), jnp.int32)
        return 0
    lax.fori_loop(0, nwords // L, body, 0)


@functools.partial(
    pl.kernel,
    mesh=_mesh,
    compiler_params=_params,
    out_type=[
        jax.ShapeDtypeStruct((N,), jnp.int32),
        jax.ShapeDtypeStruct((NW, B1 * L), jnp.int32),
    ],
    scratch_types=[
        pltpu.VMEM((CHUNK,), jnp.float32),
        pltpu.VMEM((CHUNK,), jnp.float32),
        pltpu.VMEM((CHUNK,), jnp.int32),
        pltpu.VMEM((B1 * L,), jnp.int32),
    ],
)
def _pass1(ff_hbm, fr_hbm, keys_hbm, hist_hbm, ffb, frb, keyb, hist_v):
    wid = _wid()
    base = wid * M
    lane = lax.iota(jnp.int32, L)
    ones = jnp.ones((L,), jnp.int32)
    _zero_hist(hist_v, B1 * L)

    def chunk(ci, _):
        off = base + ci * CHUNK
        pltpu.sync_copy(ff_hbm.at[pl.ds(off, CHUNK)], ffb)
        pltpu.sync_copy(fr_hbm.at[pl.ds(off, CHUNK)], frb)

        def vec(j, _):
            s = ffb[pl.ds(j * L, L)] - frb[pl.ds(j * L, L)]
            b = lax.bitcast_convert_type(s, jnp.int32)
            key = jnp.where(b >= 0, b, (-b) ^ MININT)
            keyb[pl.ds(j * L, L)] = key
            u = key ^ MININT
            d = lax.shift_right_logical(u, 20)
            plsc.addupdate_scatter(hist_v, [d * L + lane], ones)
            return 0

        lax.fori_loop(0, VECS, vec, 0)
        pltpu.sync_copy(keyb, keys_hbm.at[pl.ds(off, CHUNK)])
        return 0

    lax.fori_loop(0, NCHUNKS, chunk, 0)
    pltpu.sync_copy(hist_v, hist_hbm.at[wid])


def _make_refine(nbins, shift, prefix_shift):
    @functools.partial(
        pl.kernel,
        mesh=_mesh,
        compiler_params=_params,
        out_type=[jax.ShapeDtypeStruct((NW, nbins * L), jnp.int32)],
        scratch_types=[
            pltpu.VMEM((CHUNK,), jnp.int32),
            pltpu.VMEM((L,), jnp.int32),
            pltpu.VMEM((nbins * L,), jnp.int32),
        ],
    )
    def refine(keys_hbm, pre_hbm, hist_hbm, keyb, prev, hist_v):
        wid = _wid()
        base = wid * M
        lane = lax.iota(jnp.int32, L)
        ones = jnp.ones((L,), jnp.int32)
        _zero_hist(hist_v, nbins * L)
        pltpu.sync_copy(pre_hbm, prev)
        pre = prev[...]

        def chunk(ci, _):
            off = base + ci * CHUNK
            pltpu.sync_copy(keys_hbm.at[pl.ds(off, CHUNK)], keyb)

            def vec(j, _):
                u = keyb[pl.ds(j * L, L)] ^ MININT
                m = lax.shift_right_logical(u, prefix_shift) == pre
                d = lax.shift_right_logical(u, shift) & (nbins - 1)
                plsc.addupdate_scatter(hist_v, [d * L + lane], ones, mask=m)
                return 0

            lax.fori_loop(0, VECS, vec, 0)
            return 0

        lax.fori_loop(0, NCHUNKS, chunk, 0)
        pltpu.sync_copy(hist_v, hist_hbm.at[wid])

    return refine


_pass2 = _make_refine(B1, 8, 20)
_pass3 = _make_refine(B3, 0, 8)


@functools.partial(
    pl.kernel,
    mesh=_mesh,
    compiler_params=_params,
    out_type=[
        jax.ShapeDtypeStruct((K + 128,), jnp.int32),
        jax.ShapeDtypeStruct((NW, L), jnp.int32),
    ],
    scratch_types=[
        pltpu.VMEM((CHUNK,), jnp.int32),
        pltpu.VMEM((3 * L,), jnp.int32),
        pltpu.VMEM((160,), jnp.int32),
        pltpu.VMEM((128,), jnp.int32),
        pltpu.VMEM((L,), jnp.int32),
        pltpu.SemaphoreType.DMA,
    ],
)
def _pass4(keys_hbm, prm_hbm, out_hbm, icut_hbm, keyb, prm, packv, posv, icv, sem):
    wid = _wid()
    base = wid * M
    lane = lax.iota(jnp.int32, L)
    pltpu.sync_copy(prm_hbm.at[wid], prm)
    qv = prm[pl.ds(0, L)]
    sv = prm[pl.ds(L, L)]
    tv = prm[pl.ds(2 * L, L)]
    icv[...] = jnp.full((L,), N, jnp.int32)

    def chunk(ci, carry):
        cl, fl, tc = carry
        pltpu.sync_copy(keys_hbm.at[pl.ds(base + ci * CHUNK, CHUNK)], keyb)

        def vec(j, carry):
            cl, fl, tc = carry
            key = keyb[pl.ds(j * L, L)]
            mgt = key > tv
            meq = key == tv
            meqi = meq.astype(jnp.int32)
            lord = tc + jnp.cumsum(meqi) - 1
            msel = mgt | (meq & (lord >= qv))
            idxv = base + ci * CHUNK + j * L + lane
            plsc.store_compressed(icv.at[pl.ds(0, L)], idxv, mask=meq & (lord == qv))
            plsc.store_compressed(packv.at[pl.ds(cl, L)], idxv, mask=msel)
            cl2 = cl + jnp.sum(msel.astype(jnp.int32))
            tc2 = tc + jnp.sum(meqi)

            @pl.when(cl2 >= 128)
            def _flush():
                for jj in range(8):
                    posv[pl.ds(jj * L, L)] = sv + fl + jj * L + lane
                pltpu.async_copy(packv.at[pl.ds(0, 128)], out_hbm.at[posv], sem).wait()
                packv[pl.ds(0, L)] = packv[pl.ds(128, L)]

            did = jnp.where(cl2 >= 128, 128, 0)
            return (cl2 - did, fl + did, tc2)

        return lax.fori_loop(0, VECS, vec, (cl, fl, tc))

    cl, fl, _ = lax.fori_loop(0, NCHUNKS, chunk, (0, 0, 0))

    for jj in range(8):
        lord = jj * L + lane
        posv[pl.ds(jj * L, L)] = jnp.where(lord < cl, sv + fl + lord, K + lord)
    pltpu.async_copy(packv.at[pl.ds(0, 128)], out_hbm.at[posv], sem).wait()
    pltpu.sync_copy(icv, icut_hbm.at[wid])


def _tc_mask(w_ref, k_ref, t_ref, ic_ref, o_ref):
    t = t_ref[0]
    ic = ic_ref[0]
    r0 = pl.program_id(0) * 256
    ri = lax.broadcasted_iota(jnp.int32, (256, COLS), 0)
    ci = lax.broadcasted_iota(jnp.int32, (256, COLS), 1)
    fi = (r0 + ri) * COLS + ci
    key = k_ref[...]
    flag = (key > t) | ((key == t) & (fi >= ic))
    o_ref[...] = jnp.where(flag, 0.0, w_ref[...])


def _select_digit(h, k_rem):
    ss = jnp.cumsum(h[::-1])[::-1]
    d = jnp.sum((ss >= k_rem).astype(jnp.int32)) - 1
    ssp = jnp.concatenate([ss, jnp.zeros((1,), ss.dtype)])
    return d, k_rem - ssp[d + 1]


def kernel(W, fisher_forget, fisher_retain):
    ff = fisher_forget.reshape(-1)
    fr = fisher_retain.reshape(-1)

    keys, h1 = _pass1(ff, fr)
    h1t = h1.reshape(NW, B1, L).sum(axis=2)
    d1, k1 = _select_digit(h1t.sum(axis=0), jnp.int32(K))

    h2, = _pass2(keys, jnp.broadcast_to(d1, (L,)).astype(jnp.int32))
    h2t = h2.reshape(NW, B1, L).sum(axis=2)
    d2, k2 = _select_digit(h2t.sum(axis=0), k1)

    pre2 = (d1 << 12) | d2
    h3, = _pass3(keys, jnp.broadcast_to(pre2, (L,)).astype(jnp.int32))
    h3t = h3.reshape(NW, B3, L).sum(axis=2)
    d3, r = _select_digit(h3t.sum(axis=0), k2)

    t_key = (((d1 << 12) | d2) << 8 | d3) ^ MININT

    ar1 = jnp.arange(B1)
    ar3 = jnp.arange(B3)
    a_t = (h1t * (ar1 > d1)).sum(axis=1)
    b_t = (h2t * (ar1 > d2)).sum(axis=1)
    c_t = (h3t * (ar3 > d3)).sum(axis=1)
    ties_t = jnp.take(h3t, d3, axis=1)
    p_t = jnp.cumsum(ties_t) - ties_t
    q = ties_t.sum() - r
    d_t = jnp.clip(p_t + ties_t - q, 0, ties_t)
    s_t = a_t + b_t + c_t + d_t
    off_t = jnp.cumsum(s_t) - s_t
    prm = jnp.stack([q - p_t, off_t, jnp.full((NW,), t_key, jnp.int32)], axis=1)
    prm = jnp.broadcast_to(prm[:, :, None], (NW, 3, L)).reshape(NW, 3 * L)
    prm = prm.astype(jnp.int32)

    out_idx, icut = _pass4(keys, prm)
    mask_index = out_idx[:K]
    icut_s = jnp.min(icut)

    new_w = pl.pallas_call(
        _tc_mask,
        grid=(ROWS // 256,),
        in_specs=[
            pl.BlockSpec((256, COLS), lambda i: (i, 0)),
            pl.BlockSpec((256, COLS), lambda i: (i, 0)),
            pl.BlockSpec(memory_space=pltpu.SMEM),
            pl.BlockSpec(memory_space=pltpu.SMEM),
        ],
        out_specs=pl.BlockSpec((256, COLS), lambda i: (i, 0)),
        out_shape=jax.ShapeDtypeStruct((ROWS, COLS), jnp.float32),
    )(W, keys.reshape(ROWS, COLS), t_key.reshape(1), icut_s.reshape(1))

    return new_w, mask_index


# speedup vs baseline: 8.8321x; 1.0508x over previous
import functools

import jax
import jax.numpy as jnp
import numpy as np
from jax import lax
from jax.experimental import pallas as pl
from jax.experimental.pallas import tpu as pltpu
from jax.experimental.pallas import tpu_sc as plsc

ROWS, COLS = 8192, 4096
N = ROWS * COLS
K = N // 10
NC, NS, L = 2, 16, 16
NW = NC * NS
M = N // NW
CHUNK = 8192
VECS = CHUNK // L
NCHUNKS = M // CHUNK
B1 = 4096
B3 = 256
MININT = np.int32(-2147483648)

_mesh = plsc.VectorSubcoreMesh(core_axis_name="c", subcore_axis_name="s")
_params = pltpu.CompilerParams(needs_layout_passes=False)


def _wid():
    return lax.axis_index("s") * NC + lax.axis_index("c")


def _zero_hist(hist_v, nwords):
    def body(i, _):
        hist_v[pl.ds(i * L, L)] = jnp.zeros((L,), jnp.int32)
        return 0
    lax.fori_loop(0, nwords // L, body, 0)


@functools.partial(
    pl.kernel,
    mesh=_mesh,
    compiler_params=_params,
    out_type=[
        jax.ShapeDtypeStruct((N,), jnp.int32),
        jax.ShapeDtypeStruct((NW, B1 * L), jnp.int32),
    ],
    scratch_types=[
        pltpu.VMEM((CHUNK,), jnp.float32),
        pltpu.VMEM((CHUNK,), jnp.float32),
        pltpu.VMEM((CHUNK,), jnp.float32),
        pltpu.VMEM((CHUNK,), jnp.float32),
        pltpu.VMEM((CHUNK,), jnp.int32),
        pltpu.VMEM((CHUNK,), jnp.int32),
        pltpu.VMEM((B1 * L,), jnp.int32),
        pltpu.SemaphoreType.DMA,
        pltpu.SemaphoreType.DMA,
        pltpu.SemaphoreType.DMA,
    ],
)
def _pass1(ff_hbm, fr_hbm, keys_hbm, hist_hbm, ff0, ff1, fr0, fr1, kb0, kb1,
           hist_v, s0, s1, sw):
    wid = _wid()
    base = wid * M
    lane = lax.iota(jnp.int32, L)
    ones = jnp.ones((L,), jnp.int32)
    _zero_hist(hist_v, B1 * L)

    def rng(ci):
        return pl.ds(base + ci * CHUNK, CHUNK)

    def process(ffb, frb, keyb):
        def vec(j, _):
            s = ffb[pl.ds(j * L, L)] - frb[pl.ds(j * L, L)]
            b = lax.bitcast_convert_type(s, jnp.int32)
            key = jnp.where(b >= 0, b, (-b) ^ MININT)
            keyb[pl.ds(j * L, L)] = key
            u = key ^ MININT
            d = lax.shift_right_logical(u, 20)
            plsc.addupdate_scatter(hist_v, [d * L + lane], ones)
            return 0

        lax.fori_loop(0, VECS, vec, 0)

    pltpu.async_copy(ff_hbm.at[rng(0)], ff0, s0)
    pltpu.async_copy(fr_hbm.at[rng(0)], fr0, s0)

    def pair(i, _):
        c0 = 2 * i
        pltpu.async_copy(ff_hbm.at[rng(c0 + 1)], ff1, s1)
        pltpu.async_copy(fr_hbm.at[rng(c0 + 1)], fr1, s1)
        pltpu.make_async_copy(ff_hbm.at[rng(c0)], ff0, s0).wait()
        pltpu.make_async_copy(fr_hbm.at[rng(c0)], fr0, s0).wait()

        @pl.when(i > 0)
        def _():
            pltpu.make_async_copy(kb0, keys_hbm.at[rng(c0 - 2)], sw).wait()

        process(ff0, fr0, kb0)
        pltpu.async_copy(kb0, keys_hbm.at[rng(c0)], sw)

        @pl.when(i < NCHUNKS // 2 - 1)
        def _():
            pltpu.async_copy(ff_hbm.at[rng(c0 + 2)], ff0, s0)
            pltpu.async_copy(fr_hbm.at[rng(c0 + 2)], fr0, s0)

        pltpu.make_async_copy(ff_hbm.at[rng(c0 + 1)], ff1, s1).wait()
        pltpu.make_async_copy(fr_hbm.at[rng(c0 + 1)], fr1, s1).wait()

        @pl.when(i > 0)
        def _():
            pltpu.make_async_copy(kb1, keys_hbm.at[rng(c0 - 1)], sw).wait()

        process(ff1, fr1, kb1)
        pltpu.async_copy(kb1, keys_hbm.at[rng(c0 + 1)], sw)
        return 0

    lax.fori_loop(0, NCHUNKS // 2, pair, 0)
    pltpu.make_async_copy(kb0, keys_hbm.at[rng(NCHUNKS - 2)], sw).wait()
    pltpu.make_async_copy(kb1, keys_hbm.at[rng(NCHUNKS - 1)], sw).wait()
    pltpu.sync_copy(hist_v, hist_hbm.at[wid])


def _make_refine(nbins, shift, prefix_shift):
    @functools.partial(
        pl.kernel,
        mesh=_mesh,
        compiler_params=_params,
        out_type=[jax.ShapeDtypeStruct((NW, nbins * L), jnp.int32)],
        scratch_types=[
            pltpu.VMEM((CHUNK,), jnp.int32),
            pltpu.VMEM((CHUNK,), jnp.int32),
            pltpu.VMEM((L,), jnp.int32),
            pltpu.VMEM((nbins * L,), jnp.int32),
            pltpu.SemaphoreType.DMA,
            pltpu.SemaphoreType.DMA,
        ],
    )
    def refine(keys_hbm, pre_hbm, hist_hbm, kb0, kb1, prev, hist_v, s0, s1):
        wid = _wid()
        base = wid * M
        lane = lax.iota(jnp.int32, L)
        ones = jnp.ones((L,), jnp.int32)
        _zero_hist(hist_v, nbins * L)
        pltpu.sync_copy(pre_hbm, prev)
        pre = prev[...]

        def process(kb):
            def vec(j, _):
                u = kb[pl.ds(j * L, L)] ^ MININT
                m = lax.shift_right_logical(u, prefix_shift) == pre
                d = lax.shift_right_logical(u, shift) & (nbins - 1)
                plsc.addupdate_scatter(hist_v, [d * L + lane], ones, mask=m)
                return 0

            lax.fori_loop(0, VECS, vec, 0)

        def src(ci):
            return keys_hbm.at[pl.ds(base + ci * CHUNK, CHUNK)]

        pltpu.async_copy(src(0), kb0, s0)

        def pair(i, _):
            c0 = 2 * i
            pltpu.async_copy(src(c0 + 1), kb1, s1)
            pltpu.make_async_copy(src(c0), kb0, s0).wait()
            process(kb0)

            @pl.when(i < NCHUNKS // 2 - 1)
            def _():
                pltpu.async_copy(src(c0 + 2), kb0, s0)

            pltpu.make_async_copy(src(c0 + 1), kb1, s1).wait()
            process(kb1)
            return 0

        lax.fori_loop(0, NCHUNKS // 2, pair, 0)
        pltpu.sync_copy(hist_v, hist_hbm.at[wid])

    return refine


_pass2 = _make_refine(B1, 8, 20)
_pass3 = _make_refine(B3, 0, 8)


@functools.partial(
    pl.kernel,
    mesh=_mesh,
    compiler_params=_params,
    out_type=[
        jax.ShapeDtypeStruct((K + 128,), jnp.int32),
        jax.ShapeDtypeStruct((NW, L), jnp.int32),
    ],
    scratch_types=[
        pltpu.VMEM((CHUNK,), jnp.int32),
        pltpu.VMEM((3 * L,), jnp.int32),
        pltpu.VMEM((160,), jnp.int32),
        pltpu.VMEM((128,), jnp.int32),
        pltpu.VMEM((L,), jnp.int32),
        pltpu.SemaphoreType.DMA,
    ],
)
def _pass4(keys_hbm, prm_hbm, out_hbm, icut_hbm, keyb, prm, packv, posv, icv, sem):
    wid = _wid()
    base = wid * M
    lane = lax.iota(jnp.int32, L)
    pltpu.sync_copy(prm_hbm.at[wid], prm)
    qv = prm[pl.ds(0, L)]
    sv = prm[pl.ds(L, L)]
    tv = prm[pl.ds(2 * L, L)]
    icv[...] = jnp.full((L,), N, jnp.int32)

    def chunk(ci, carry):
        cl, fl, tc = carry
        pltpu.sync_copy(keys_hbm.at[pl.ds(base + ci * CHUNK, CHUNK)], keyb)

        def vec(j, carry):
            cl, fl, tc = carry
            key = keyb[pl.ds(j * L, L)]
            mgt = key > tv
            meq = key == tv
            meqi = meq.astype(jnp.int32)
            lord = tc + jnp.cumsum(meqi) - 1
            msel = mgt | (meq & (lord >= qv))
            idxv = base + ci * CHUNK + j * L + lane
            plsc.store_compressed(icv.at[pl.ds(0, L)], idxv, mask=meq & (lord == qv))
            plsc.store_compressed(packv.at[pl.ds(cl, L)], idxv, mask=msel)
            cl2 = cl + jnp.sum(msel.astype(jnp.int32))
            tc2 = tc + jnp.sum(meqi)

            @pl.when(cl2 >= 128)
            def _flush():
                for jj in range(8):
                    posv[pl.ds(jj * L, L)] = sv + fl + jj * L + lane
                pltpu.async_copy(packv.at[pl.ds(0, 128)], out_hbm.at[posv], sem).wait()
                packv[pl.ds(0, L)] = packv[pl.ds(128, L)]

            did = jnp.where(cl2 >= 128, 128, 0)
            return (cl2 - did, fl + did, tc2)

        return lax.fori_loop(0, VECS, vec, (cl, fl, tc))

    cl, fl, _ = lax.fori_loop(0, NCHUNKS, chunk, (0, 0, 0))

    for jj in range(8):
        lord = jj * L + lane
        posv[pl.ds(jj * L, L)] = jnp.where(lord < cl, sv + fl + lord, K + lord)
    pltpu.async_copy(packv.at[pl.ds(0, 128)], out_hbm.at[posv], sem).wait()
    pltpu.sync_copy(icv, icut_hbm.at[wid])


def _tc_mask(w_ref, k_ref, t_ref, ic_ref, o_ref):
    t = t_ref[0]
    ic = ic_ref[0]
    r0 = pl.program_id(0) * 256
    ri = lax.broadcasted_iota(jnp.int32, (256, COLS), 0)
    ci = lax.broadcasted_iota(jnp.int32, (256, COLS), 1)
    fi = (r0 + ri) * COLS + ci
    key = k_ref[...]
    flag = (key > t) | ((key == t) & (fi >= ic))
    o_ref[...] = jnp.where(flag, 0.0, w_ref[...])


def _select_digit(h, k_rem):
    ss = jnp.cumsum(h[::-1])[::-1]
    d = jnp.sum((ss >= k_rem).astype(jnp.int32)) - 1
    ssp = jnp.concatenate([ss, jnp.zeros((1,), ss.dtype)])
    return d, k_rem - ssp[d + 1]


def kernel(W, fisher_forget, fisher_retain):
    ff = fisher_forget.reshape(-1)
    fr = fisher_retain.reshape(-1)

    keys, h1 = _pass1(ff, fr)
    h1t = h1.reshape(NW, B1, L).sum(axis=2)
    d1, k1 = _select_digit(h1t.sum(axis=0), jnp.int32(K))

    h2, = _pass2(keys, jnp.broadcast_to(d1, (L,)).astype(jnp.int32))
    h2t = h2.reshape(NW, B1, L).sum(axis=2)
    d2, k2 = _select_digit(h2t.sum(axis=0), k1)

    pre2 = (d1 << 12) | d2
    h3, = _pass3(keys, jnp.broadcast_to(pre2, (L,)).astype(jnp.int32))
    h3t = h3.reshape(NW, B3, L).sum(axis=2)
    d3, r = _select_digit(h3t.sum(axis=0), k2)

    t_key = (((d1 << 12) | d2) << 8 | d3) ^ MININT

    ar1 = jnp.arange(B1)
    ar3 = jnp.arange(B3)
    a_t = (h1t * (ar1 > d1)).sum(axis=1)
    b_t = (h2t * (ar1 > d2)).sum(axis=1)
    c_t = (h3t * (ar3 > d3)).sum(axis=1)
    ties_t = jnp.take(h3t, d3, axis=1)
    p_t = jnp.cumsum(ties_t) - ties_t
    q = ties_t.sum() - r
    d_t = jnp.clip(p_t + ties_t - q, 0, ties_t)
    s_t = a_t + b_t + c_t + d_t
    off_t = jnp.cumsum(s_t) - s_t
    prm = jnp.stack([q - p_t, off_t, jnp.full((NW,), t_key, jnp.int32)], axis=1)
    prm = jnp.broadcast_to(prm[:, :, None], (NW, 3, L)).reshape(NW, 3 * L)
    prm = prm.astype(jnp.int32)

    out_idx, icut = _pass4(keys, prm)
    mask_index = out_idx[:K]
    icut_s = jnp.min(icut)

    new_w = pl.pallas_call(
        _tc_mask,
        grid=(ROWS // 256,),
        in_specs=[
            pl.BlockSpec((256, COLS), lambda i: (i, 0)),
            pl.BlockSpec((256, COLS), lambda i: (i, 0)),
            pl.BlockSpec(memory_space=pltpu.SMEM),
            pl.BlockSpec(memory_space=pltpu.SMEM),
        ],
        out_specs=pl.BlockSpec((256, COLS), lambda i: (i, 0)),
        out_shape=jax.ShapeDtypeStruct((ROWS, COLS), jnp.float32),
    )(W, keys.reshape(ROWS, COLS), t_key.reshape(1), icut_s.reshape(1))

    return new_w, mask_index
